# baseline (device time: 18027 ns/iter reference)
import jax
import jax.numpy as jnp
from jax import lax
from jax.experimental import pallas as pl
from jax.experimental.pallas import tpu as pltpu

N_DEV = 4
EPS = 1e-5
B = 2


def kernel(x, gamma, beta):
    m, n_loc = x.shape
    n_glob = n_loc * N_DEV
    mb = m // B

    def body(x_hbm, g_ref, b_ref, o_hbm, xb, ob, comm_ref,
             load_sems, store_sems, send_sems, recv_sems):
        my = lax.axis_index("i")

        loads = [
            pltpu.make_async_copy(
                x_hbm.at[pl.ds(i * mb, mb), :], xb.at[i], load_sems.at[i])
            for i in range(B)
        ]
        for ld in loads:
            ld.start()

        barrier_sem = pltpu.get_barrier_semaphore()
        for d in range(1, N_DEV):
            pl.semaphore_signal(
                barrier_sem, inc=1,
                device_id=(lax.rem(my + d, N_DEV),),
                device_id_type=pl.DeviceIdType.MESH,
            )

        ones_row = jnp.ones((1, n_loc), jnp.float32)
        nt = (((1,), (1,)), ((), ()))

        pl.semaphore_wait(barrier_sem, N_DEV - 1)

        rdmas = {}
        for i in range(B):
            loads[i].wait()
            xv = xb[i]
            comm_ref[0, i, 0:1, :] = lax.dot_general(
                ones_row, xv, nt, preferred_element_type=jnp.float32)
            comm_ref[0, i, 1:2, :] = lax.dot_general(
                ones_row, xv * xv, nt, preferred_element_type=jnp.float32)
            for d in range(1, N_DEV):
                r = pltpu.make_async_remote_copy(
                    src_ref=comm_ref.at[0, i],
                    dst_ref=comm_ref.at[d, i],
                    send_sem=send_sems.at[i, d - 1],
                    recv_sem=recv_sems.at[i, d - 1],
                    device_id=(lax.rem(my + d, N_DEV),),
                    device_id_type=pl.DeviceIdType.MESH,
                )
                r.start()
                rdmas[(i, d)] = r

        stores = []
        for i in range(B):
            for d in range(1, N_DEV):
                rdmas[(i, d)].wait()
            tot = (comm_ref[0, i] + comm_ref[1, i]
                   + comm_ref[2, i] + comm_ref[3, i])
            tot_c = tot.T
            mean = tot_c[:, 0:1] / n_glob
            var = tot_c[:, 1:2] / n_glob - mean * mean
            inv = lax.rsqrt(var + EPS)
            ob[i] = g_ref[:, :] * (xb[i] - mean) * inv + b_ref[:, :]
            s = pltpu.make_async_copy(
                ob.at[i], o_hbm.at[pl.ds(i * mb, mb), :], store_sems.at[i])
            s.start()
            stores.append(s)
        for s in stores:
            s.wait()

    return pl.pallas_call(
        body,
        out_shape=jax.ShapeDtypeStruct((m, n_loc), jnp.float32),
        in_specs=[
            pl.BlockSpec(memory_space=pl.ANY),
            pl.BlockSpec(memory_space=pltpu.VMEM),
            pl.BlockSpec(memory_space=pltpu.VMEM),
        ],
        out_specs=pl.BlockSpec(memory_space=pl.ANY),
        scratch_shapes=[
            pltpu.VMEM((B, mb, n_loc), jnp.float32),
            pltpu.VMEM((B, mb, n_loc), jnp.float32),
            pltpu.VMEM((N_DEV, B, 2, mb), jnp.float32),
            pltpu.SemaphoreType.DMA((B,)),
            pltpu.SemaphoreType.DMA((B,)),
            pltpu.SemaphoreType.DMA((B, N_DEV - 1)),
            pltpu.SemaphoreType.DMA((B, N_DEV - 1)),
        ],
        compiler_params=pltpu.CompilerParams(collective_id=0),
    )(x, gamma.reshape(1, n_loc), beta.reshape(1, n_loc))


# device time: 16752 ns/iter; 1.0761x vs baseline; 1.0761x over previous
import jax
import jax.numpy as jnp
from jax import lax
from jax.experimental import pallas as pl
from jax.experimental.pallas import tpu as pltpu

N_DEV = 4
EPS = 1e-5


def kernel(x, gamma, beta):
    m, n_loc = x.shape
    n_glob = n_loc * N_DEV

    def body(x_ref, g_ref, b_ref, o_ref, comm_ref, send_sems, recv_sems):
        my = lax.axis_index("i")

        barrier_sem = pltpu.get_barrier_semaphore()
        for d in range(1, N_DEV):
            pl.semaphore_signal(
                barrier_sem, inc=1,
                device_id=(lax.rem(my + d, N_DEV),),
                device_id_type=pl.DeviceIdType.MESH,
            )
        xv = x_ref[:, :]
        s1 = jnp.sum(xv, axis=1, keepdims=True)
        s2 = jnp.sum(xv * xv, axis=1, keepdims=True)
        comm_ref[0, :, :] = jnp.concatenate([s1, s2], axis=1).T

        pl.semaphore_wait(barrier_sem, N_DEV - 1)

        rdmas = []
        for d in range(1, N_DEV):
            rdma = pltpu.make_async_remote_copy(
                src_ref=comm_ref.at[0],
                dst_ref=comm_ref.at[d],
                send_sem=send_sems.at[d - 1],
                recv_sem=recv_sems.at[d - 1],
                device_id=(lax.rem(my + d, N_DEV),),
                device_id_type=pl.DeviceIdType.MESH,
            )
            rdma.start()
            rdmas.append(rdma)
        for rdma in rdmas:
            rdma.wait()

        tot = (comm_ref[0, :, :] + comm_ref[1, :, :]
               + comm_ref[2, :, :] + comm_ref[3, :, :])
        tot_c = tot.T
        mean = tot_c[:, 0:1] / n_glob
        var = tot_c[:, 1:2] / n_glob - mean * mean
        inv = lax.rsqrt(var + EPS)
        o_ref[:, :] = g_ref[:, :] * (xv - mean) * inv + b_ref[:, :]

    return pl.pallas_call(
        body,
        out_shape=jax.ShapeDtypeStruct((m, n_loc), jnp.float32),
        in_specs=[
            pl.BlockSpec(memory_space=pltpu.VMEM),
            pl.BlockSpec(memory_space=pltpu.VMEM),
            pl.BlockSpec(memory_space=pltpu.VMEM),
        ],
        out_specs=pl.BlockSpec(memory_space=pltpu.VMEM),
        scratch_shapes=[
            pltpu.VMEM((N_DEV, 2, m), jnp.float32),
            pltpu.SemaphoreType.DMA((N_DEV - 1,)),
            pltpu.SemaphoreType.DMA((N_DEV - 1,)),
        ],
        compiler_params=pltpu.CompilerParams(collective_id=0),
    )(x, gamma.reshape(1, n_loc), beta.reshape(1, n_loc))
